# baseline (speedup 1.0000x reference)
import functools

import jax
import jax.numpy as jnp
from jax.experimental import pallas as pl
from jax.experimental.pallas import tpu as pltpu

B, N, D_NODE = 16, 128, 128
HID1, HID2 = 128, 256
EDGE_HID = 128
NK = 10
KC = 12800 // NK


def _gnn_kernel(nf_ref, ef_ref, W1_ref, b1_ref, W2_ref, b2_ref,
                We_ref, be_ref, Wcn_ref, Wce_ref, bc_ref, out_ref, acc_ref):
    k = pl.program_id(0)

    @pl.when(k == 0)
    def _init():
        acc_ref[...] = jnp.zeros_like(acc_ref)

    acc_ref[...] += jnp.dot(ef_ref[...], We_ref[...],
                            preferred_element_type=jnp.float32)

    @pl.when(k == NK - 1)
    def _finish():
        m = jnp.mean(nf_ref[...], axis=1)
        e1 = jax.nn.relu(
            jnp.dot(m, W1_ref[...], preferred_element_type=jnp.float32)
            + b1_ref[...])
        e2 = jax.nn.relu(
            jnp.dot(e1, W2_ref[...], preferred_element_type=jnp.float32)
            + b2_ref[...])
        ee = jax.nn.relu(acc_ref[...] + be_ref[...])
        out_ref[...] = (
            jnp.dot(e2, Wcn_ref[...], preferred_element_type=jnp.float32)
            + jnp.dot(ee, Wce_ref[...], preferred_element_type=jnp.float32)
            + bc_ref[...])


def kernel(node_features, edge_features, W1, b1, W2, b2, We, be, Wc, bc):
    b = node_features.shape[0]
    ef_flat = edge_features.reshape(b, -1)
    full = lambda *shape: pl.BlockSpec(shape, lambda k: (0,) * len(shape))
    out = pl.pallas_call(
        _gnn_kernel,
        grid=(NK,),
        in_specs=[
            full(b, N, D_NODE),
            pl.BlockSpec((b, KC), lambda k: (0, k)),
            full(D_NODE, HID1), full(1, HID1),
            full(HID1, HID2), full(1, HID2),
            pl.BlockSpec((KC, EDGE_HID), lambda k: (k, 0)),
            full(1, EDGE_HID),
            full(HID2, Wc.shape[1]), full(EDGE_HID, Wc.shape[1]),
            full(1, Wc.shape[1]),
        ],
        out_specs=pl.BlockSpec((b, Wc.shape[1]), lambda k: (0, 0)),
        out_shape=jax.ShapeDtypeStruct((b, Wc.shape[1]), jnp.float32),
        scratch_shapes=[pltpu.VMEM((b, EDGE_HID), jnp.float32)],
        compiler_params=pltpu.CompilerParams(
            dimension_semantics=("arbitrary",)),
    )(node_features, ef_flat,
      W1, b1.reshape(1, -1), W2, b2.reshape(1, -1),
      We, be.reshape(1, -1),
      Wc[:HID2], Wc[HID2:], bc.reshape(1, -1))
    return out


# speedup vs baseline: 1.3053x; 1.3053x over previous
import jax
import jax.numpy as jnp
from jax.experimental import pallas as pl
from jax.experimental.pallas import tpu as pltpu

B, N, D_NODE = 16, 128, 128
HID1, HID2 = 128, 256
EDGE_HID = 128
K_EDGE = 12800
NCHUNK = 8
KC = K_EDGE // NCHUNK


def _gnn_kernel(nf_ref, ef_ref, W1_ref, b1_ref, W2_ref, b2_ref,
                We_hbm, be_ref, Wcn_ref, Wce_ref, bc_ref, out_ref,
                ws_ref, sems):
    copies = [
        pltpu.make_async_copy(
            We_hbm.at[pl.ds(c * KC, KC), :],
            ws_ref.at[pl.ds(c * KC, KC), :],
            sems.at[c])
        for c in range(NCHUNK)
    ]
    for cp in copies:
        cp.start()

    m = jnp.mean(nf_ref[...], axis=1)
    e1 = jax.nn.relu(
        jnp.dot(m, W1_ref[...], preferred_element_type=jnp.float32)
        + b1_ref[...])
    e2 = jax.nn.relu(
        jnp.dot(e1, W2_ref[...], preferred_element_type=jnp.float32)
        + b2_ref[...])

    acc = jnp.zeros((B, EDGE_HID), dtype=jnp.float32)
    for c, cp in enumerate(copies):
        cp.wait()
        acc += jnp.dot(ef_ref[:, c * KC:(c + 1) * KC],
                       ws_ref[pl.ds(c * KC, KC), :],
                       preferred_element_type=jnp.float32)
    ee = jax.nn.relu(acc + be_ref[...])

    out_ref[...] = (
        jnp.dot(e2, Wcn_ref[...], preferred_element_type=jnp.float32)
        + jnp.dot(ee, Wce_ref[...], preferred_element_type=jnp.float32)
        + bc_ref[...])


def kernel(node_features, edge_features, W1, b1, W2, b2, We, be, Wc, bc):
    b = node_features.shape[0]
    ef_flat = edge_features.reshape(b, -1)
    vmem = pl.BlockSpec(memory_space=pltpu.MemorySpace.VMEM)
    out = pl.pallas_call(
        _gnn_kernel,
        in_specs=[vmem, vmem, vmem, vmem, vmem, vmem,
                  pl.BlockSpec(memory_space=pltpu.MemorySpace.HBM),
                  vmem, vmem, vmem, vmem],
        out_specs=vmem,
        out_shape=jax.ShapeDtypeStruct((b, Wc.shape[1]), jnp.float32),
        scratch_shapes=[
            pltpu.VMEM((K_EDGE, EDGE_HID), jnp.float32),
            pltpu.SemaphoreType.DMA((NCHUNK,)),
        ],
    )(node_features, ef_flat,
      W1, b1.reshape(1, -1), W2, b2.reshape(1, -1),
      We, be.reshape(1, -1),
      Wc[:HID2], Wc[HID2:], bc.reshape(1, -1))
    return out


# speedup vs baseline: 2.1196x; 1.6239x over previous
import jax
import jax.numpy as jnp
from jax.experimental import pallas as pl

B, N, D_NODE = 16, 128, 128
HID1, HID2 = 128, 256
EDGE_HID = 128


def _gnn_kernel(nf_ref, ef_ref, W1_ref, b1_ref, W2_ref, b2_ref,
                We_ref, be_ref, Wc_ref, bc_ref, out_ref):
    m = jnp.mean(nf_ref[...], axis=1)
    e1 = jax.nn.relu(
        jnp.dot(m, W1_ref[...], preferred_element_type=jnp.float32)
        + b1_ref[...])
    e2 = jax.nn.relu(
        jnp.dot(e1, W2_ref[...], preferred_element_type=jnp.float32)
        + b2_ref[...])
    ee = jax.nn.relu(
        jnp.dot(ef_ref[...], We_ref[...], preferred_element_type=jnp.float32)
        + be_ref[...])
    out_ref[...] = (
        jnp.dot(e2, Wc_ref[0:HID2, :], preferred_element_type=jnp.float32)
        + jnp.dot(ee, Wc_ref[HID2:HID2 + EDGE_HID, :],
                  preferred_element_type=jnp.float32)
        + bc_ref[...])


def kernel(node_features, edge_features, W1, b1, W2, b2, We, be, Wc, bc):
    b = node_features.shape[0]
    ef_flat = edge_features.reshape(b, -1)
    out = pl.pallas_call(
        _gnn_kernel,
        out_shape=jax.ShapeDtypeStruct((b, Wc.shape[1]), jnp.float32),
    )(node_features, ef_flat,
      W1, b1.reshape(1, -1), W2, b2.reshape(1, -1),
      We, be.reshape(1, -1),
      Wc, bc.reshape(1, -1))
    return out
